# baseline (speedup 1.0000x reference)
import jax
import jax.numpy as jnp
from jax import lax
from jax.experimental import pallas as pl
from jax.experimental.pallas import tpu as pltpu
from jax.experimental.pallas import tpu_sc as plsc

N = 10000
E = 320000
D = 128
DH = D // 2

NC = 2
NS = 16
CHUNK = 128
NPAD = 10240
EPAD = 327680
EPT = EPAD // NS
NCH = EPT // CHUNK
DEG_NCH = NCH // NC
ROWS_PER_TILE = NPAD // NS
ZROWS = 128
LANES = 16

_MESH = plsc.VectorSubcoreMesh(core_axis_name="c", subcore_axis_name="s")


def _sc_aggregate_body(h_hbm, row_hbm, col_hbm, s_out, deg_out,
                       row_v, col_v, gbuf0, gbuf1, ones_v, zbuf, zdbuf,
                       s_sh, deg_sh, sem_g0, sem_g1):
    cid = lax.axis_index("c")
    sid = lax.axis_index("s")

    zeros16 = jnp.zeros((LANES,), jnp.float32)
    ones16 = jnp.ones((LANES,), jnp.float32)

    @pl.loop(0, CHUNK)
    def _(i):
        ones_v[i, :] = ones16

    @pl.loop(0, ZROWS)
    def _(i):
        zdbuf[i, :] = zeros16

        @pl.loop(0, DH, step=LANES)
        def _(j):
            zbuf[i, pl.ds(j, LANES)] = zeros16

    @pl.loop(0, ROWS_PER_TILE, step=ZROWS)
    def _(k):
        pltpu.sync_copy(zbuf, s_sh.at[pl.ds(sid * ROWS_PER_TILE + k, ZROWS)])
        pltpu.sync_copy(zdbuf, deg_sh.at[pl.ds(sid * ROWS_PER_TILE + k, ZROWS)])

    pltpu.sync_copy(row_hbm.at[sid], row_v)
    pltpu.sync_copy(col_hbm.at[sid], col_v)

    plsc.subcore_barrier()

    hc = h_hbm.at[cid]
    dbase = cid * DEG_NCH

    pltpu.async_copy(hc.at[row_v.at[0]], gbuf0, sem_g0)
    pltpu.async_copy(hc.at[row_v.at[1]], gbuf1, sem_g1)

    @pl.loop(0, NCH, step=2)
    def _(j):
        dj = dbase + lax.div(j, 2)
        pltpu.sync_copy(ones_v, deg_sh.at[row_v.at[dj]], add=True)

        pltpu.make_async_copy(hc.at[row_v.at[j]], gbuf0, sem_g0).wait()
        pltpu.sync_copy(gbuf0, s_sh.at[col_v.at[j]], add=True)

        @pl.when(j + 2 < NCH)
        def _():
            pltpu.async_copy(hc.at[row_v.at[j + 2]], gbuf0, sem_g0)

        pltpu.make_async_copy(hc.at[row_v.at[j + 1]], gbuf1, sem_g1).wait()
        pltpu.sync_copy(gbuf1, s_sh.at[col_v.at[j + 1]], add=True)

        @pl.when(j + 3 < NCH)
        def _():
            pltpu.async_copy(hc.at[row_v.at[j + 3]], gbuf1, sem_g1)

    plsc.subcore_barrier()

    base = sid * ROWS_PER_TILE
    pltpu.sync_copy(s_sh.at[pl.ds(base, ROWS_PER_TILE)],
                    s_out.at[cid, pl.ds(base, ROWS_PER_TILE)])
    pltpu.sync_copy(deg_sh.at[pl.ds(base, ROWS_PER_TILE)],
                    deg_out.at[cid, pl.ds(base, ROWS_PER_TILE)])


@jax.jit
def _sc_aggregate(h2, row3d, col3d):
    kern = pl.kernel(
        _sc_aggregate_body,
        out_type=(
            jax.ShapeDtypeStruct((NC, NPAD, DH), jnp.float32),
            jax.ShapeDtypeStruct((NC, NPAD, LANES), jnp.float32),
        ),
        mesh=_MESH,
        scratch_types=[
            pltpu.VMEM((NCH, CHUNK), jnp.int32),
            pltpu.VMEM((NCH, CHUNK), jnp.int32),
            pltpu.VMEM((CHUNK, DH), jnp.float32),
            pltpu.VMEM((CHUNK, DH), jnp.float32),
            pltpu.VMEM((CHUNK, LANES), jnp.float32),
            pltpu.VMEM((ZROWS, DH), jnp.float32),
            pltpu.VMEM((ZROWS, LANES), jnp.float32),
            pltpu.VMEM_SHARED((NPAD, DH), jnp.float32),
            pltpu.VMEM_SHARED((NPAD, LANES), jnp.float32),
            pltpu.SemaphoreType.DMA,
            pltpu.SemaphoreType.DMA,
        ],
        compiler_params=pltpu.CompilerParams(use_tc_tiling_on_sc=False),
    )
    return kern(h2, row3d, col3d)


def _tc_linear1_body(x_ref, w_ref, b_ref, o_ref):
    h = jnp.dot(x_ref[...], w_ref[...], preferred_element_type=jnp.float32)
    h = jnp.maximum(h + b_ref[...], 0.0)
    o_ref[0] = h[:, :DH]
    o_ref[1] = h[:, DH:]


@jax.jit
def _tc_linear1(x, w1t, b1):
    bm = 400
    return pl.pallas_call(
        _tc_linear1_body,
        grid=(N // bm,),
        in_specs=[
            pl.BlockSpec((bm, D), lambda i: (i, 0)),
            pl.BlockSpec((D, D), lambda i: (0, 0)),
            pl.BlockSpec((1, D), lambda i: (0, 0)),
        ],
        out_specs=pl.BlockSpec((NC, bm, DH), lambda i: (0, i, 0)),
        out_shape=jax.ShapeDtypeStruct((NC, NPAD, DH), jnp.float32),
    )(x, w1t, b1)


def _tc_linear2_body(s_ref, d_ref, w_ref, b_ref, o_ref):
    s = jnp.concatenate([s_ref[0], s_ref[1]], axis=1)
    deg = d_ref[0][:, 0:1] + d_ref[1][:, 0:1]
    dinv = jnp.where(deg > 0.0, 1.0 / deg, 0.0)
    agg = s * dinv
    o_ref[...] = (
        jnp.dot(agg, w_ref[...], preferred_element_type=jnp.float32)
        + b_ref[...]
    )


@jax.jit
def _tc_linear2(s_part, deg_part, w2t, b2):
    bm = 400
    return pl.pallas_call(
        _tc_linear2_body,
        grid=(N // bm,),
        in_specs=[
            pl.BlockSpec((NC, bm, DH), lambda i: (0, i, 0)),
            pl.BlockSpec((NC, bm, LANES), lambda i: (0, i, 0)),
            pl.BlockSpec((D, D), lambda i: (0, 0)),
            pl.BlockSpec((1, D), lambda i: (0, 0)),
        ],
        out_specs=pl.BlockSpec((bm, D), lambda i: (i, 0)),
        out_shape=jax.ShapeDtypeStruct((N, D), jnp.float32),
    )(s_part, deg_part, w2t, b2)


@jax.jit
def kernel(x, edge_index, W1, b1, W2, b2):
    epad = jnp.full((2, EPAD - E), N, jnp.int32)
    ei = jnp.concatenate([edge_index, epad], axis=1)
    row3d = ei[0].reshape(NS, NCH, CHUNK)
    col3d = ei[1].reshape(NS, NCH, CHUNK)

    h2 = _tc_linear1(x, W1.T, b1.reshape(1, D))
    s_part, deg_part = _sc_aggregate(h2, row3d, col3d)
    out = _tc_linear2(s_part, deg_part, W2.T, b2.reshape(1, D))
    return out


# speedup vs baseline: 1.7046x; 1.7046x over previous
import jax
import jax.numpy as jnp
from jax import lax
from jax.experimental import pallas as pl
from jax.experimental.pallas import tpu as pltpu
from jax.experimental.pallas import tpu_sc as plsc

N = 10000
E = 320000
D = 128
DH = D // 2

NC = 2
NS = 16
CHUNK = 128
NPAD = 10240
EPAD = 327680
EPT = EPAD // NS
NCH = EPT // CHUNK
DEG_NCH = NCH // NC
ROWS_PER_TILE = NPAD // NS
ZROWS = 128
LANES = 16

_MESH = plsc.VectorSubcoreMesh(core_axis_name="c", subcore_axis_name="s")


def _sc_aggregate_body(h_hbm, row_hbm, col_hbm, s_out, deg_out,
                       row_v, col_v, gbuf0, gbuf1, ones_v, zbuf, zdbuf,
                       s_sh, deg_sh, sem_g0, sem_g1):
    cid = lax.axis_index("c")
    sid = lax.axis_index("s")

    zeros16 = jnp.zeros((LANES,), jnp.float32)
    ones16 = jnp.ones((LANES,), jnp.float32)

    @pl.loop(0, CHUNK)
    def _(i):
        ones_v[i, :] = ones16

    @pl.loop(0, ZROWS)
    def _(i):
        zdbuf[i, :] = zeros16

        @pl.loop(0, DH, step=LANES)
        def _(j):
            zbuf[i, pl.ds(j, LANES)] = zeros16

    @pl.loop(0, ROWS_PER_TILE, step=ZROWS)
    def _(k):
        pltpu.sync_copy(zbuf, s_sh.at[pl.ds(sid * ROWS_PER_TILE + k, ZROWS)])
        pltpu.sync_copy(zdbuf, deg_sh.at[pl.ds(sid * ROWS_PER_TILE + k, ZROWS)])

    pltpu.sync_copy(row_hbm.at[sid], row_v)
    pltpu.sync_copy(col_hbm.at[sid], col_v)

    plsc.subcore_barrier()

    hc = h_hbm.at[cid]
    dbase = cid * DEG_NCH

    pltpu.async_copy(hc.at[row_v.at[0]], gbuf0, sem_g0)
    pltpu.async_copy(hc.at[row_v.at[1]], gbuf1, sem_g1)

    @pl.loop(0, NCH, step=2)
    def _(j):
        dj = dbase + lax.div(j, 2)
        pltpu.sync_copy(ones_v, deg_sh.at[row_v.at[dj]], add=True)

        pltpu.make_async_copy(hc.at[row_v.at[j]], gbuf0, sem_g0).wait()
        pltpu.sync_copy(gbuf0, s_sh.at[col_v.at[j]], add=True)

        @pl.when(j + 2 < NCH)
        def _():
            pltpu.async_copy(hc.at[row_v.at[j + 2]], gbuf0, sem_g0)

        pltpu.make_async_copy(hc.at[row_v.at[j + 1]], gbuf1, sem_g1).wait()
        pltpu.sync_copy(gbuf1, s_sh.at[col_v.at[j + 1]], add=True)

        @pl.when(j + 3 < NCH)
        def _():
            pltpu.async_copy(hc.at[row_v.at[j + 3]], gbuf1, sem_g1)

    plsc.subcore_barrier()

    base = sid * ROWS_PER_TILE
    pltpu.sync_copy(s_sh.at[pl.ds(base, ROWS_PER_TILE)],
                    s_out.at[cid, pl.ds(base, ROWS_PER_TILE)])
    pltpu.sync_copy(deg_sh.at[pl.ds(base, ROWS_PER_TILE)],
                    deg_out.at[cid, pl.ds(base, ROWS_PER_TILE)])


@jax.jit
def _sc_aggregate(h2, row3d, col3d):
    kern = pl.kernel(
        _sc_aggregate_body,
        out_type=(
            jax.ShapeDtypeStruct((NC, NPAD, DH), jnp.float32),
            jax.ShapeDtypeStruct((NC, NPAD, LANES), jnp.float32),
        ),
        mesh=_MESH,
        scratch_types=[
            pltpu.VMEM((NCH, CHUNK), jnp.int32),
            pltpu.VMEM((NCH, CHUNK), jnp.int32),
            pltpu.VMEM((CHUNK, DH), jnp.float32),
            pltpu.VMEM((CHUNK, DH), jnp.float32),
            pltpu.VMEM((CHUNK, LANES), jnp.float32),
            pltpu.VMEM((ZROWS, DH), jnp.float32),
            pltpu.VMEM((ZROWS, LANES), jnp.float32),
            pltpu.VMEM_SHARED((NPAD, DH), jnp.float32),
            pltpu.VMEM_SHARED((NPAD, LANES), jnp.float32),
            pltpu.SemaphoreType.DMA,
            pltpu.SemaphoreType.DMA,
        ],
        compiler_params=pltpu.CompilerParams(use_tc_tiling_on_sc=False),
    )
    return kern(h2, row3d, col3d)


def _tc_linear1_body(x_ref, w_ref, b_ref, o_ref):
    h = jnp.dot(x_ref[...], w_ref[...], preferred_element_type=jnp.float32)
    h = jnp.maximum(h + b_ref[...], 0.0)
    o_ref[0] = h[:, :DH]
    o_ref[1] = h[:, DH:]


@jax.jit
def _tc_linear1(x, w1t, b1):
    bm = 400
    return pl.pallas_call(
        _tc_linear1_body,
        grid=(N // bm,),
        in_specs=[
            pl.BlockSpec((bm, D), lambda i: (i, 0)),
            pl.BlockSpec((D, D), lambda i: (0, 0)),
            pl.BlockSpec((1, D), lambda i: (0, 0)),
        ],
        out_specs=pl.BlockSpec((NC, bm, DH), lambda i: (0, i, 0)),
        out_shape=jax.ShapeDtypeStruct((NC, NPAD, DH), jnp.float32),
    )(x, w1t, b1)


def _tc_linear2_body(s_ref, d_ref, w_ref, b_ref, o_ref):
    s = jnp.concatenate([s_ref[0], s_ref[1]], axis=1)
    deg = d_ref[0][:, 0:1] + d_ref[1][:, 0:1]
    dinv = jnp.where(deg > 0.0, 1.0 / deg, 0.0)
    agg = s * dinv
    o_ref[...] = (
        jnp.dot(agg, w_ref[...], preferred_element_type=jnp.float32)
        + b_ref[...]
    )


@jax.jit
def _tc_linear2(s_part, deg_part, w2t, b2):
    bm = 400
    return pl.pallas_call(
        _tc_linear2_body,
        grid=(N // bm,),
        in_specs=[
            pl.BlockSpec((NC, bm, DH), lambda i: (0, i, 0)),
            pl.BlockSpec((NC, bm, LANES), lambda i: (0, i, 0)),
            pl.BlockSpec((D, D), lambda i: (0, 0)),
            pl.BlockSpec((1, D), lambda i: (0, 0)),
        ],
        out_specs=pl.BlockSpec((bm, D), lambda i: (i, 0)),
        out_shape=jax.ShapeDtypeStruct((N, D), jnp.float32),
    )(s_part, deg_part, w2t, b2)


@jax.jit
def kernel(x, edge_index, W1, b1, W2, b2):
    pad_ids = N + jnp.arange(EPAD - E, dtype=jnp.int32) % (NPAD - N)
    epad = jnp.broadcast_to(pad_ids, (2, EPAD - E))
    ei = jnp.concatenate([edge_index, epad], axis=1)
    row3d = ei[0].reshape(NS, NCH, CHUNK)
    col3d = ei[1].reshape(NS, NCH, CHUNK)

    h2 = _tc_linear1(x, W1.T, b1.reshape(1, D))
    s_part, deg_part = _sc_aggregate(h2, row3d, col3d)
    out = _tc_linear2(s_part, deg_part, W2.T, b2.reshape(1, D))
    return out


# speedup vs baseline: 1.7759x; 1.0418x over previous
import jax
import jax.numpy as jnp
from jax import lax
from jax.experimental import pallas as pl
from jax.experimental.pallas import tpu as pltpu
from jax.experimental.pallas import tpu_sc as plsc

N = 10000
E = 320000
D = 128
DH = D // 2

NC = 2
NS = 16
CHUNK = 128
NPAD = 10240
EPAD = 327680
EPT = EPAD // NS
NCH = EPT // CHUNK
DEG_NCH = NCH // NC
ROWS_PER_TILE = NPAD // NS
ZROWS = 128
LANES = 16

_MESH = plsc.VectorSubcoreMesh(core_axis_name="c", subcore_axis_name="s")


def _sc_aggregate_body(h_hbm, row_hbm, col_hbm, s_out, deg_out,
                       row_v, col_v, gbuf0, gbuf1, ones_v, zbuf, zdbuf,
                       s_sh, deg_sh, sem_g0, sem_g1):
    cid = lax.axis_index("c")
    sid = lax.axis_index("s")

    zeros16 = jnp.zeros((LANES,), jnp.float32)
    ones16 = jnp.ones((LANES,), jnp.float32)

    @pl.loop(0, CHUNK)
    def _(i):
        ones_v[i, :] = ones16

    @pl.loop(0, ZROWS)
    def _(i):
        zdbuf[i, :] = zeros16

        @pl.loop(0, DH, step=LANES)
        def _(j):
            zbuf[i, pl.ds(j, LANES)] = zeros16

    @pl.loop(0, ROWS_PER_TILE, step=ZROWS)
    def _(k):
        pltpu.sync_copy(zbuf, s_sh.at[pl.ds(sid * ROWS_PER_TILE + k, ZROWS)])
        pltpu.sync_copy(zdbuf, deg_sh.at[pl.ds(sid * ROWS_PER_TILE + k, ZROWS)])

    pltpu.sync_copy(row_hbm.at[sid], row_v)
    pltpu.sync_copy(col_hbm.at[sid], col_v)

    plsc.subcore_barrier()

    hc = h_hbm.at[cid]
    dbase = cid * DEG_NCH

    pltpu.async_copy(hc.at[row_v.at[0]], gbuf0, sem_g0)
    pltpu.async_copy(hc.at[row_v.at[1]], gbuf1, sem_g1)

    @pl.loop(0, NCH, step=2)
    def _(j):
        dj = dbase + lax.div(j, 2)
        pltpu.sync_copy(ones_v, deg_sh.at[row_v.at[dj]], add=True)

        pltpu.make_async_copy(hc.at[row_v.at[j]], gbuf0, sem_g0).wait()
        pltpu.sync_copy(gbuf0, s_sh.at[col_v.at[j]], add=True)

        @pl.when(j + 2 < NCH)
        def _():
            pltpu.async_copy(hc.at[row_v.at[j + 2]], gbuf0, sem_g0)

        pltpu.make_async_copy(hc.at[row_v.at[j + 1]], gbuf1, sem_g1).wait()
        pltpu.sync_copy(gbuf1, s_sh.at[col_v.at[j + 1]], add=True)

        @pl.when(j + 3 < NCH)
        def _():
            pltpu.async_copy(hc.at[row_v.at[j + 3]], gbuf1, sem_g1)

    plsc.subcore_barrier()

    base = sid * ROWS_PER_TILE
    pltpu.sync_copy(s_sh.at[pl.ds(base, ROWS_PER_TILE)],
                    s_out.at[pl.ds(base, ROWS_PER_TILE), pl.ds(cid * DH, DH)])
    pltpu.sync_copy(deg_sh.at[pl.ds(base, ROWS_PER_TILE)],
                    deg_out.at[cid, pl.ds(base, ROWS_PER_TILE)])


@jax.jit
def _sc_aggregate(h2, row3d, col3d):
    kern = pl.kernel(
        _sc_aggregate_body,
        out_type=(
            jax.ShapeDtypeStruct((NPAD, D), jnp.float32),
            jax.ShapeDtypeStruct((NC, NPAD, LANES), jnp.float32),
        ),
        mesh=_MESH,
        scratch_types=[
            pltpu.VMEM((NCH, CHUNK), jnp.int32),
            pltpu.VMEM((NCH, CHUNK), jnp.int32),
            pltpu.VMEM((CHUNK, DH), jnp.float32),
            pltpu.VMEM((CHUNK, DH), jnp.float32),
            pltpu.VMEM((CHUNK, LANES), jnp.float32),
            pltpu.VMEM((ZROWS, DH), jnp.float32),
            pltpu.VMEM((ZROWS, LANES), jnp.float32),
            pltpu.VMEM_SHARED((NPAD, DH), jnp.float32),
            pltpu.VMEM_SHARED((NPAD, LANES), jnp.float32),
            pltpu.SemaphoreType.DMA,
            pltpu.SemaphoreType.DMA,
        ],
        compiler_params=pltpu.CompilerParams(use_tc_tiling_on_sc=False),
    )
    return kern(h2, row3d, col3d)


def _tc_linear1_body(x_ref, w_ref, b_ref, o_ref):
    h = jnp.dot(x_ref[...], w_ref[...], preferred_element_type=jnp.float32)
    h = jnp.maximum(h + b_ref[...], 0.0)
    o_ref[0] = h[:, :DH]
    o_ref[1] = h[:, DH:]


@jax.jit
def _tc_linear1(x, w1t, b1):
    bm = 400
    return pl.pallas_call(
        _tc_linear1_body,
        grid=(N // bm,),
        in_specs=[
            pl.BlockSpec((bm, D), lambda i: (i, 0)),
            pl.BlockSpec((D, D), lambda i: (0, 0)),
            pl.BlockSpec((1, D), lambda i: (0, 0)),
        ],
        out_specs=pl.BlockSpec((NC, bm, DH), lambda i: (0, i, 0)),
        out_shape=jax.ShapeDtypeStruct((NC, NPAD, DH), jnp.float32),
    )(x, w1t, b1)


def _tc_linear2_body(s_ref, d_ref, w_ref, b_ref, o_ref):
    deg = d_ref[0][:, 0:1] + d_ref[1][:, 0:1]
    dinv = jnp.where(deg > 0.0, 1.0 / deg, 0.0)
    agg = s_ref[...] * dinv
    o_ref[...] = (
        jnp.dot(agg, w_ref[...], preferred_element_type=jnp.float32)
        + b_ref[...]
    )


@jax.jit
def _tc_linear2(s_part, deg_part, w2t, b2):
    bm = 400
    return pl.pallas_call(
        _tc_linear2_body,
        grid=(N // bm,),
        in_specs=[
            pl.BlockSpec((bm, D), lambda i: (i, 0)),
            pl.BlockSpec((NC, bm, LANES), lambda i: (0, i, 0)),
            pl.BlockSpec((D, D), lambda i: (0, 0)),
            pl.BlockSpec((1, D), lambda i: (0, 0)),
        ],
        out_specs=pl.BlockSpec((bm, D), lambda i: (i, 0)),
        out_shape=jax.ShapeDtypeStruct((N, D), jnp.float32),
    )(s_part, deg_part, w2t, b2)


@jax.jit
def kernel(x, edge_index, W1, b1, W2, b2):
    pad_ids = N + jnp.arange(EPAD - E, dtype=jnp.int32) % (NPAD - N)
    epad = jnp.broadcast_to(pad_ids, (2, EPAD - E))
    ei = jnp.concatenate([edge_index, epad], axis=1)
    row3d = ei[0].reshape(NS, NCH, CHUNK)
    col3d = ei[1].reshape(NS, NCH, CHUNK)

    h2 = _tc_linear1(x, W1.T, b1.reshape(1, D))
    s_part, deg_part = _sc_aggregate(h2, row3d, col3d)

    out = _tc_linear2(s_part, deg_part, W2.T, b2.reshape(1, D))
    return out


# speedup vs baseline: 1.8224x; 1.0262x over previous
import jax
import jax.numpy as jnp
from jax import lax
from jax.experimental import pallas as pl
from jax.experimental.pallas import tpu as pltpu
from jax.experimental.pallas import tpu_sc as plsc

N = 10000
E = 320000
D = 128
DH = D // 2

NC = 2
NS = 16
CHUNK = 128
NPAD = 10240
EPAD = 327680
EPT = EPAD // NS
NCH = EPT // CHUNK
DEG_NCH = NCH // NC
ROWS_PER_TILE = NPAD // NS
ZROWS = 128
LANES = 16

_MESH = plsc.VectorSubcoreMesh(core_axis_name="c", subcore_axis_name="s")


def _sc_aggregate_body(h_hbm, row_hbm, col_hbm, s_out, deg_out,
                       row_v, col_v, gbuf0, gbuf1, gbuf2, gbuf3,
                       zbuf, hist_v, s_sh,
                       sem_g0, sem_g1, sem_g2, sem_g3):
    cid = lax.axis_index("c")
    sid = lax.axis_index("s")

    zeros16 = jnp.zeros((LANES,), jnp.float32)
    ones16 = jnp.ones((LANES,), jnp.float32)

    @pl.loop(0, ZROWS)
    def _(i):
        @pl.loop(0, DH, step=LANES)
        def _(j):
            zbuf[i, pl.ds(j, LANES)] = zeros16

    @pl.loop(0, NPAD, step=LANES)
    def _(i):
        hist_v[pl.ds(i, LANES)] = zeros16

    @pl.loop(0, ROWS_PER_TILE, step=ZROWS)
    def _(k):
        pltpu.sync_copy(zbuf, s_sh.at[pl.ds(sid * ROWS_PER_TILE + k, ZROWS)])

    pltpu.sync_copy(row_hbm.at[sid], row_v)
    pltpu.sync_copy(col_hbm.at[sid], col_v)

    plsc.subcore_barrier()

    hc = h_hbm.at[cid]
    dbase = cid * DEG_NCH

    bufs = (gbuf0, gbuf1)
    sems = (sem_g0, sem_g1)
    NB = len(bufs)

    for k in range(NB):
        pltpu.async_copy(hc.at[row_v.at[k]], bufs[k], sems[k])

    @pl.loop(0, NCH, step=NB)
    def _(j):
        for k in range(NB):
            if k % 2 == 0:
                dj = dbase + lax.div(j, 2) + k // 2

                @pl.loop(0, CHUNK, step=LANES)
                def _(q):
                    idx = row_v[dj, pl.ds(q, LANES)]
                    plsc.addupdate_scatter(hist_v, [idx], ones16)

            pltpu.make_async_copy(hc.at[row_v.at[j + k]], bufs[k],
                                  sems[k]).wait()
            pltpu.sync_copy(bufs[k], s_sh.at[col_v.at[j + k]], add=True)

            @pl.when(j + NB + k < NCH)
            def _():
                pltpu.async_copy(hc.at[row_v.at[j + NB + k]], bufs[k], sems[k])

    plsc.subcore_barrier()

    base = sid * ROWS_PER_TILE
    pltpu.sync_copy(s_sh.at[pl.ds(base, ROWS_PER_TILE)],
                    s_out.at[pl.ds(base, ROWS_PER_TILE), pl.ds(cid * DH, DH)])
    pltpu.sync_copy(hist_v, deg_out.at[cid * NS + sid])


@jax.jit
def _sc_aggregate(h2, row3d, col3d):
    kern = pl.kernel(
        _sc_aggregate_body,
        out_type=(
            jax.ShapeDtypeStruct((NPAD, D), jnp.float32),
            jax.ShapeDtypeStruct((NC * NS, NPAD), jnp.float32),
        ),
        mesh=_MESH,
        scratch_types=[
            pltpu.VMEM((NCH, CHUNK), jnp.int32),
            pltpu.VMEM((NCH, CHUNK), jnp.int32),
            pltpu.VMEM((CHUNK, DH), jnp.float32),
            pltpu.VMEM((CHUNK, DH), jnp.float32),
            pltpu.VMEM((CHUNK, DH), jnp.float32),
            pltpu.VMEM((CHUNK, DH), jnp.float32),
            pltpu.VMEM((ZROWS, DH), jnp.float32),
            pltpu.VMEM((NPAD,), jnp.float32),
            pltpu.VMEM_SHARED((NPAD, DH), jnp.float32),
            pltpu.SemaphoreType.DMA,
            pltpu.SemaphoreType.DMA,
            pltpu.SemaphoreType.DMA,
            pltpu.SemaphoreType.DMA,
        ],
        compiler_params=pltpu.CompilerParams(use_tc_tiling_on_sc=False,
                                             needs_layout_passes=False),
    )
    return kern(h2, row3d, col3d)


def _tc_linear1_body(x_ref, w_ref, b_ref, o_ref):
    h = jnp.dot(x_ref[...], w_ref[...], preferred_element_type=jnp.float32)
    h = jnp.maximum(h + b_ref[...], 0.0)
    o_ref[0] = h[:, :DH]
    o_ref[1] = h[:, DH:]


@jax.jit
def _tc_linear1(x, w1t, b1):
    bm = 400
    return pl.pallas_call(
        _tc_linear1_body,
        grid=(N // bm,),
        in_specs=[
            pl.BlockSpec((bm, D), lambda i: (i, 0)),
            pl.BlockSpec((D, D), lambda i: (0, 0)),
            pl.BlockSpec((1, D), lambda i: (0, 0)),
        ],
        out_specs=pl.BlockSpec((NC, bm, DH), lambda i: (0, i, 0)),
        out_shape=jax.ShapeDtypeStruct((NC, NPAD, DH), jnp.float32),
    )(x, w1t, b1)


def _tc_linear2_body(s_ref, d_ref, w_ref, b_ref, o_ref):
    deg = jnp.sum(d_ref[...], axis=0)[:, None]
    dinv = jnp.where(deg > 0.0, 1.0 / deg, 0.0)
    agg = s_ref[...] * dinv
    o_ref[...] = (
        jnp.dot(agg, w_ref[...], preferred_element_type=jnp.float32)
        + b_ref[...]
    )


@jax.jit
def _tc_linear2(s_part, deg_part, w2t, b2):
    bm = 512
    return pl.pallas_call(
        _tc_linear2_body,
        grid=(NPAD // bm,),
        in_specs=[
            pl.BlockSpec((bm, D), lambda i: (i, 0)),
            pl.BlockSpec((NC * NS, bm), lambda i: (0, i)),
            pl.BlockSpec((D, D), lambda i: (0, 0)),
            pl.BlockSpec((1, D), lambda i: (0, 0)),
        ],
        out_specs=pl.BlockSpec((bm, D), lambda i: (i, 0)),
        out_shape=jax.ShapeDtypeStruct((NPAD, D), jnp.float32),
    )(s_part, deg_part, w2t, b2)


@jax.jit
def kernel(x, edge_index, W1, b1, W2, b2):
    pad_ids = N + jnp.arange(EPAD - E, dtype=jnp.int32) % (NPAD - N)
    epad = jnp.broadcast_to(pad_ids, (2, EPAD - E))
    ei = jnp.concatenate([edge_index, epad], axis=1)
    row3d = ei[0].reshape(NS, NCH, CHUNK)
    col3d = ei[1].reshape(NS, NCH, CHUNK)

    h2 = _tc_linear1(x, W1.T, b1.reshape(1, D))
    s_part, deg_part = _sc_aggregate(h2, row3d, col3d)

    out = _tc_linear2(s_part, deg_part, W2.T, b2.reshape(1, D))
    return out[:N]


# speedup vs baseline: 1.9049x; 1.0452x over previous
import jax
import jax.numpy as jnp
from jax import lax
from jax.experimental import pallas as pl
from jax.experimental.pallas import tpu as pltpu
from jax.experimental.pallas import tpu_sc as plsc

N = 10000
E = 320000
D = 128
DH = D // 2

NC = 2
NS = 16
CHUNK = 128
NPAD = 10240
EPAD = 327680
EPT = EPAD // NS
NCH = EPT // CHUNK
DEG_NCH = NCH // NC
ROWS_PER_TILE = NPAD // NS
ZROWS = 128
LANES = 16

_MESH = plsc.VectorSubcoreMesh(core_axis_name="c", subcore_axis_name="s")


def _sc_aggregate_body(h_hbm, row_hbm, col_hbm, s_out, deg_out,
                       row_v, col_v, gbuf0, gbuf1, gbuf2, gbuf3,
                       zbuf, hist_v, s_sh,
                       sem_g0, sem_g1, sem_g2, sem_g3):
    cid = lax.axis_index("c")
    sid = lax.axis_index("s")

    zeros16 = jnp.zeros((LANES,), jnp.float32)
    ones16 = jnp.ones((LANES,), jnp.float32)

    @pl.loop(0, ZROWS)
    def _(i):
        @pl.loop(0, DH, step=LANES)
        def _(j):
            zbuf[i, pl.ds(j, LANES)] = zeros16

    @pl.loop(0, NPAD, step=LANES)
    def _(i):
        hist_v[pl.ds(i, LANES)] = zeros16

    @pl.loop(0, ROWS_PER_TILE, step=ZROWS)
    def _(k):
        pltpu.sync_copy(zbuf, s_sh.at[pl.ds(sid * ROWS_PER_TILE + k, ZROWS)])

    pltpu.sync_copy(row_hbm.at[cid, sid], row_v)
    pltpu.sync_copy(col_hbm.at[sid], col_v)

    plsc.subcore_barrier()

    hc = h_hbm
    dbase = cid * DEG_NCH

    bufs = (gbuf0, gbuf1)
    sems = (sem_g0, sem_g1)
    NB = len(bufs)

    for k in range(NB):
        pltpu.async_copy(hc.at[row_v.at[k]], bufs[k], sems[k])

    @pl.loop(0, NCH, step=NB)
    def _(j):
        for k in range(NB):
            if k % 2 == 0:
                dj = dbase + lax.div(j, 2) + k // 2

                @pl.loop(0, CHUNK, step=LANES)
                def _(q):
                    idx = lax.shift_right_logical(row_v[dj, pl.ds(q, LANES)],
                                                  1)
                    plsc.addupdate_scatter(hist_v, [idx], ones16)

            pltpu.make_async_copy(hc.at[row_v.at[j + k]], bufs[k],
                                  sems[k]).wait()
            pltpu.sync_copy(bufs[k], s_sh.at[col_v.at[j + k]], add=True)

            @pl.when(j + NB + k < NCH)
            def _():
                pltpu.async_copy(hc.at[row_v.at[j + NB + k]], bufs[k], sems[k])

    plsc.subcore_barrier()

    base = sid * ROWS_PER_TILE
    pltpu.sync_copy(s_sh.at[pl.ds(base, ROWS_PER_TILE)],
                    s_out.at[pl.ds(base, ROWS_PER_TILE), pl.ds(cid * DH, DH)])
    pltpu.sync_copy(hist_v, deg_out.at[cid * NS + sid])


@jax.jit
def _sc_aggregate(h2, row3d, col3d):
    kern = pl.kernel(
        _sc_aggregate_body,
        out_type=(
            jax.ShapeDtypeStruct((NPAD, D), jnp.float32),
            jax.ShapeDtypeStruct((NC * NS, NPAD), jnp.float32),
        ),
        mesh=_MESH,
        scratch_types=[
            pltpu.VMEM((NCH, CHUNK), jnp.int32),
            pltpu.VMEM((NCH, CHUNK), jnp.int32),
            pltpu.VMEM((CHUNK, DH), jnp.float32),
            pltpu.VMEM((CHUNK, DH), jnp.float32),
            pltpu.VMEM((CHUNK, DH), jnp.float32),
            pltpu.VMEM((CHUNK, DH), jnp.float32),
            pltpu.VMEM((ZROWS, DH), jnp.float32),
            pltpu.VMEM((NPAD,), jnp.float32),
            pltpu.VMEM_SHARED((NPAD, DH), jnp.float32),
            pltpu.SemaphoreType.DMA,
            pltpu.SemaphoreType.DMA,
            pltpu.SemaphoreType.DMA,
            pltpu.SemaphoreType.DMA,
        ],
        compiler_params=pltpu.CompilerParams(use_tc_tiling_on_sc=False,
                                             needs_layout_passes=False),
    )
    return kern(h2, row3d, col3d)


def _tc_linear1_body(x_ref, w_ref, b_ref, o_ref):
    h = jnp.dot(x_ref[...], w_ref[...], preferred_element_type=jnp.float32)
    o_ref[...] = jnp.maximum(h + b_ref[...], 0.0)


@jax.jit
def _tc_linear1(x, w1t, b1):
    bm = 400
    return pl.pallas_call(
        _tc_linear1_body,
        grid=(N // bm,),
        in_specs=[
            pl.BlockSpec((bm, D), lambda i: (i, 0)),
            pl.BlockSpec((D, D), lambda i: (0, 0)),
            pl.BlockSpec((1, D), lambda i: (0, 0)),
        ],
        out_specs=pl.BlockSpec((bm, D), lambda i: (i, 0)),
        out_shape=jax.ShapeDtypeStruct((NPAD, D), jnp.float32),
    )(x, w1t, b1)


def _tc_linear2_body(s_ref, d_ref, w_ref, b_ref, o_ref):
    deg = jnp.sum(d_ref[...], axis=0)[:, None]
    dinv = jnp.where(deg > 0.0, 1.0 / deg, 0.0)
    agg = s_ref[...] * dinv
    o_ref[...] = (
        jnp.dot(agg, w_ref[...], preferred_element_type=jnp.float32)
        + b_ref[...]
    )


@jax.jit
def _tc_linear2(s_part, deg_part, w2t, b2):
    bm = 512
    return pl.pallas_call(
        _tc_linear2_body,
        grid=(NPAD // bm,),
        in_specs=[
            pl.BlockSpec((bm, D), lambda i: (i, 0)),
            pl.BlockSpec((NC * NS, bm), lambda i: (0, i)),
            pl.BlockSpec((D, D), lambda i: (0, 0)),
            pl.BlockSpec((1, D), lambda i: (0, 0)),
        ],
        out_specs=pl.BlockSpec((bm, D), lambda i: (i, 0)),
        out_shape=jax.ShapeDtypeStruct((NPAD, D), jnp.float32),
    )(s_part, deg_part, w2t, b2)


@jax.jit
def kernel(x, edge_index, W1, b1, W2, b2):
    pad_ids = N + jnp.arange(EPAD - E, dtype=jnp.int32) % (NPAD - N)
    epad = jnp.broadcast_to(pad_ids, (2, EPAD - E))
    ei = jnp.concatenate([edge_index, epad], axis=1)
    row2 = ei[0] * 2
    rowg = jnp.stack([row2, row2 + 1]).reshape(NC, NS, NCH, CHUNK)
    col3d = ei[1].reshape(NS, NCH, CHUNK)

    h = _tc_linear1(x, W1.T, b1.reshape(1, D))
    h2 = h.reshape(NC * NPAD, DH)
    s_part, deg_part = _sc_aggregate(h2, rowg, col3d)

    out = _tc_linear2(s_part, deg_part, W2.T, b2.reshape(1, D))
    return out[:N]
